# Initial kernel scaffold; baseline (speedup 1.0000x reference)
import jax
import jax.numpy as jnp
from jax.experimental import pallas as pl


def kernel(x, edge_index, Wf0, Wf1, Wf2, bf, Wb0, Wb1, Wb2, bb, Wl0, Wl1, Wl2, bl, g1, be1, g2, be2, g3, be3, p, Wg, bg):
    raise NotImplementedError("write your pallas kernel here")


# speedup vs baseline: 5.5498x; 5.5498x over previous
import functools

import jax
import jax.numpy as jnp
from jax import lax
from jax.experimental import pallas as pl
from jax.experimental.pallas import tpu as pltpu
from jax.experimental.pallas import tpu_sc as plsc

N = 10000
E = 320000
H = 128
EMB = 128
EPS = 1e-5
NS = 16
NC = 2
RPT = 624
TAIL0 = NS * RPT
TAIL = N - TAIL0
CHUNK = 80
EPT = E // (NC * NS)
BLK = 2000
NBLK = N // BLK

_mesh = plsc.VectorSubcoreMesh(core_axis_name="c", subcore_axis_name="s")


def _rows_copy(sid, src, dst, soff=0, doff=0):
    b = sid * RPT
    pltpu.sync_copy(src.at[pl.ds(soff + b, RPT)], dst.at[pl.ds(doff + b, RPT)])

    @pl.when(sid == 0)
    def _():
        pltpu.sync_copy(src.at[pl.ds(soff + TAIL0, TAIL)],
                        dst.at[pl.ds(doff + TAIL0, TAIL)])


def _make_prop(nparts):
    @functools.partial(
        pl.kernel,
        out_type=[jax.ShapeDtypeStruct((2 * N, 128), jnp.float32)
                  for _ in range(nparts)],
        mesh=_mesh,
        scratch_types=[
            pltpu.VMEM((CHUNK,), jnp.int32),
            pltpu.VMEM((CHUNK,), jnp.int32),
            pltpu.VMEM((CHUNK, 128), jnp.float32),
            pltpu.VMEM_SHARED((N, 128), jnp.float32),
            pltpu.SemaphoreType.DMA,
        ],
    )
    def prop(*refs):
        ts = refs[:nparts]
        zeros_hbm = refs[nparts]
        rows_hbm = refs[nparts + 1]
        cols_hbm = refs[nparts + 2]
        qs = refs[nparts + 3:nparts + 3 + nparts]
        rowbuf, colbuf, gbuf, acc, sem = refs[nparts + 3 + nparts:]
        cid = lax.axis_index("c")
        sid = lax.axis_index("s")
        base = cid * (E // NC) + sid * EPT

        for t_hbm, q_hbm in zip(ts, qs):
            @pl.when(cid == 0)
            def _():
                _rows_copy(sid, t_hbm, acc)

            @pl.when(cid == 1)
            def _():
                _rows_copy(sid, zeros_hbm, acc)

            plsc.subcore_barrier()

            def body(j, carry):
                off = base + j * CHUNK
                pltpu.sync_copy(rows_hbm.at[pl.ds(off, CHUNK)], rowbuf)
                pltpu.sync_copy(cols_hbm.at[pl.ds(off, CHUNK)], colbuf)
                pltpu.async_copy(t_hbm.at[rowbuf], gbuf, sem).wait()
                pltpu.sync_copy(gbuf, acc.at[colbuf], add=True)
                return carry

            lax.fori_loop(0, EPT // CHUNK, body, 0)
            plsc.subcore_barrier()
            _rows_copy(sid, acc, q_hbm, doff=cid * N)

    return prop


_prop1 = _make_prop(1)
_prop3 = _make_prop(3)


def _row_spec(cols):
    return pl.BlockSpec((BLK, cols), lambda i: (i, 0))


def _full_spec(r, c):
    return pl.BlockSpec((r, c), lambda i: (0, 0))


def _prep_body(qa_ref, qb_ref, x_ref, deg_ref, t1_ref):
    deg = qa_ref[...] + qb_ref[...]
    deg_ref[...] = deg[:, :16]
    t1_ref[...] = x_ref[...] * lax.rsqrt(deg[:, :1])


def _prep_call(qa, qb, x):
    return pl.pallas_call(
        _prep_body,
        grid=(NBLK,),
        in_specs=[_row_spec(128), _row_spec(128), _row_spec(128)],
        out_specs=[_row_spec(16), _row_spec(128)],
        out_shape=[jax.ShapeDtypeStruct((N, 16), jnp.float32),
                   jax.ShapeDtypeStruct((N, 128), jnp.float32)],
    )(qa, qb, x)


def _make_scale_body(nparts):
    def body(*refs):
        deg_ref = refs[2 * nparts]
        p_refs = refs[2 * nparts + 1:2 * nparts + 1 + nparts]
        t_refs = refs[2 * nparts + 1 + nparts:]
        r = 1.0 / deg_ref[:, :1]
        for k in range(nparts):
            psum = refs[2 * k][...] + refs[2 * k + 1][...]
            p_refs[k][...] = psum
            t_refs[k][...] = psum * r
    return body


def _scale_call(qparts, deg):
    nparts = len(qparts)
    flat = [a for pair in qparts for a in pair]
    shp = jax.ShapeDtypeStruct((N, 128), jnp.float32)
    out = pl.pallas_call(
        _make_scale_body(nparts),
        grid=(NBLK,),
        in_specs=[_row_spec(128)] * (2 * nparts) + [_row_spec(16)],
        out_specs=[_row_spec(128)] * (2 * nparts),
        out_shape=[shp] * (2 * nparts),
    )(*flat, deg)
    return out[:nparts], out[nparts:]


def _make_mm_body(first, nparts):
    def body(*refs):
        nh = 1 if first else nparts
        h0p = refs[:nh]
        p1p = refs[nh:nh + nparts]
        q2p = refs[nh + nparts:nh + nparts + 2 * nparts]
        deg_ref = refs[nh + 3 * nparts]
        w0_ref, w1_ref, w2_ref, b_ref = refs[nh + 3 * nparts + 1:
                                             nh + 3 * nparts + 5]
        m_ref, stats_ref = refs[nh + 3 * nparts + 5:]
        i = pl.program_id(0)
        deg = deg_ref[:, :1]
        dis = lax.rsqrt(deg)
        if first:
            h0 = h0p[0][...]
        else:
            h0 = jnp.concatenate([r[...] for r in h0p], axis=1) * jnp.sqrt(deg)
        p1 = jnp.concatenate([r[...] for r in p1p], axis=1) * dis
        p2 = jnp.concatenate(
            [q2p[2 * k][...] + q2p[2 * k + 1][...] for k in range(nparts)],
            axis=1) * dis
        dn = (((1,), (1,)), ((), ()))
        m0 = lax.dot_general(h0, w0_ref[...], dn,
                             preferred_element_type=jnp.float32)
        m1 = lax.dot_general(p1, w1_ref[...], dn,
                             preferred_element_type=jnp.float32)
        m2 = lax.dot_general(p2, w2_ref[...], dn,
                             preferred_element_type=jnp.float32)
        m = jnp.concatenate([m0, m1, m2], axis=1) + b_ref[...]
        m_ref[...] = m

        @pl.when(i == 0)
        def _():
            stats_ref[...] = jnp.zeros_like(stats_ref)

        stats_ref[0:1, :] += jnp.sum(m, axis=0, keepdims=True)
        stats_ref[1:2, :] += jnp.sum(m * m, axis=0, keepdims=True)

    return body


def _mm_call(first, h0parts, p1parts, q2parts, deg, w0, w1, w2, b):
    nparts = len(p1parts)
    din = w0.shape[1]
    q2flat = [a for pair in q2parts for a in pair]
    nin = len(h0parts) + nparts + 2 * nparts
    return pl.pallas_call(
        _make_mm_body(first, nparts),
        grid=(NBLK,),
        in_specs=[_row_spec(128)] * nin + [_row_spec(16)] +
                 [_full_spec(H, din)] * 3 + [_full_spec(1, 3 * H)],
        out_specs=[_row_spec(3 * H), _full_spec(8, 3 * H)],
        out_shape=[jax.ShapeDtypeStruct((N, 3 * H), jnp.float32),
                   jax.ShapeDtypeStruct((8, 3 * H), jnp.float32)],
    )(*h0parts, *p1parts, *q2flat, deg, w0, w1, w2, b.reshape(1, -1))


def _bn_body(m_ref, stats_ref, deg_ref, g_ref, be_ref, t0_ref, t1_ref, t2_ref):
    mu = stats_ref[0:1, :] * (1.0 / N)
    var = stats_ref[1:2, :] * (1.0 / N) - mu * mu
    y = (m_ref[...] - mu) * lax.rsqrt(var + EPS) * g_ref[...] + be_ref[...]
    y = jnp.where(y >= 0.0, y, 0.1 * y)
    t = y * lax.rsqrt(deg_ref[:, :1])
    t0_ref[...] = t[:, 0:128]
    t1_ref[...] = t[:, 128:256]
    t2_ref[...] = t[:, 256:384]


def _bn_call(m, stats, deg, g, be):
    shp = jax.ShapeDtypeStruct((N, 128), jnp.float32)
    return pl.pallas_call(
        _bn_body,
        grid=(NBLK,),
        in_specs=[_row_spec(3 * H), _full_spec(8, 3 * H), _row_spec(16),
                  _full_spec(1, 3 * H), _full_spec(1, 3 * H)],
        out_specs=[_row_spec(128)] * 3,
        out_shape=[shp] * 3,
    )(m, stats, deg, g.reshape(1, -1), be.reshape(1, -1))


def _safe_pow(x, p):
    safe = jnp.where(x > 0.0, x, 1.0)
    return jnp.where(x > 0.0, jnp.exp(p * jnp.log(safe)), 0.0)


def _final_body(m_ref, stats_ref, g_ref, be_ref, p_ref, wg_ref, bg_ref,
                logits_ref, ypred_ref, psum_ref):
    i = pl.program_id(0)
    mu = stats_ref[0:1, :] * (1.0 / N)
    var = stats_ref[1:2, :] * (1.0 / N) - mu * mu
    y = (m_ref[...] - mu) * lax.rsqrt(var + EPS) * g_ref[...] + be_ref[...]
    y = jnp.where(y >= 0.0, y, 0.1 * y)
    pp = p_ref[0, 0]
    s = _safe_pow(jnp.clip(y, 0.0, 100.0), pp)

    @pl.when(i == 0)
    def _():
        psum_ref[...] = jnp.zeros_like(psum_ref)

    psum_ref[0:1, :] += jnp.sum(s, axis=0, keepdims=True)

    @pl.when(i == NBLK - 1)
    def _():
        mean = psum_ref[0:1, :] * (1.0 / N)
        pooled = _safe_pow(jnp.clip(mean, 0.0, 100.0), 1.0 / pp)
        dn = (((1,), (1,)), ((), ()))
        logits = lax.dot_general(pooled, wg_ref[...], dn,
                                 preferred_element_type=jnp.float32)
        logits = logits + bg_ref[...]
        logits_ref[...] = logits
        iota = lax.broadcasted_iota(jnp.int32, (1, 10), 1)
        mx = jnp.max(logits, axis=1, keepdims=True)
        ypred_ref[...] = jnp.min(jnp.where(logits == mx, iota, 10),
                                 axis=1, keepdims=True)


def _final_call(m, stats, g, be, p, wg, bg):
    return pl.pallas_call(
        _final_body,
        grid=(NBLK,),
        in_specs=[_row_spec(3 * H), _full_spec(8, 3 * H),
                  _full_spec(1, 3 * H), _full_spec(1, 3 * H),
                  _full_spec(1, 1), _full_spec(10, 3 * EMB), _full_spec(1, 10)],
        out_specs=[_full_spec(1, 10), _full_spec(1, 1)],
        out_shape=[jax.ShapeDtypeStruct((1, 10), jnp.float32),
                   jax.ShapeDtypeStruct((1, 1), jnp.int32)],
        scratch_shapes=[pltpu.VMEM((8, 3 * H), jnp.float32)],
    )(m, stats, g.reshape(1, -1), be.reshape(1, -1), p.reshape(1, 1),
      wg, bg.reshape(1, -1))


def _halves(q):
    return (q[:N], q[N:])


def _run_prop(propfn, ts, zeros_n, rows, cols):
    out = propfn(*ts, zeros_n, rows, cols)
    return list(out) if isinstance(out, (list, tuple)) else [out]


def kernel(x, edge_index, Wf0, Wf1, Wf2, bf, Wb0, Wb1, Wb2, bb,
           Wl0, Wl1, Wl2, bl, g1, be1, g2, be2, g3, be3, p, Wg, bg):
    rows = edge_index[0]
    cols = edge_index[1]
    ones_n = jnp.ones((N, 128), jnp.float32)
    zeros_n = jnp.zeros((N, 128), jnp.float32)

    [qdeg] = _run_prop(_prop1, [ones_n], zeros_n, rows, cols)
    deg, t1 = _prep_call(qdeg[:N], qdeg[N:], x)

    [q1] = _run_prop(_prop1, [t1], zeros_n, rows, cols)
    (p1,), (t2,) = _scale_call([_halves(q1)], deg)
    [q2] = _run_prop(_prop1, [t2], zeros_n, rows, cols)
    m1, s1 = _mm_call(True, [x], [p1], [_halves(q2)], deg, Wf0, Wf1, Wf2, bf)
    t3 = _bn_call(m1, s1, deg, g1, be1)

    q3 = _run_prop(_prop3, t3, zeros_n, rows, cols)
    p3, t4 = _scale_call([_halves(q) for q in q3], deg)
    q4 = _run_prop(_prop3, t4, zeros_n, rows, cols)
    m2, s2 = _mm_call(False, t3, p3, [_halves(q) for q in q4], deg,
                      Wb0, Wb1, Wb2, bb)
    t5 = _bn_call(m2, s2, deg, g2, be2)

    q5 = _run_prop(_prop3, t5, zeros_n, rows, cols)
    p5, t6 = _scale_call([_halves(q) for q in q5], deg)
    q6 = _run_prop(_prop3, t6, zeros_n, rows, cols)
    m3, s3 = _mm_call(False, t5, p5, [_halves(q) for q in q6], deg,
                      Wl0, Wl1, Wl2, bl)

    logits, ypred = _final_call(m3, s3, g3, be3, p, Wg, bg)
    return (logits, ypred.reshape(-1))


# speedup vs baseline: 9.2098x; 1.6595x over previous
import functools

import jax
import jax.numpy as jnp
from jax import lax
from jax.experimental import pallas as pl
from jax.experimental.pallas import tpu as pltpu
from jax.experimental.pallas import tpu_sc as plsc

N = 10000
E = 320000
H = 128
EMB = 128
EPS = 1e-5
NS = 16
NC = 2
RPT = 624
TAIL0 = NS * RPT
TAIL = N - TAIL0
CHUNK = 80
EPT = E // (NC * NS)
BLK = 2000
NBLK = N // BLK

_mesh = plsc.VectorSubcoreMesh(core_axis_name="c", subcore_axis_name="s")


def _rows_copy(sid, src, dst, soff=0, doff=0):
    b = sid * RPT
    pltpu.sync_copy(src.at[pl.ds(soff + b, RPT)], dst.at[pl.ds(doff + b, RPT)])

    @pl.when(sid == 0)
    def _():
        pltpu.sync_copy(src.at[pl.ds(soff + TAIL0, TAIL)],
                        dst.at[pl.ds(doff + TAIL0, TAIL)])


def _make_prop(nparts):
    nch = EPT // CHUNK

    @functools.partial(
        pl.kernel,
        out_type=[jax.ShapeDtypeStruct((2 * N, 128), jnp.float32)
                  for _ in range(nparts)],
        mesh=_mesh,
        scratch_types=[
            pltpu.VMEM((CHUNK,), jnp.int32),
            pltpu.VMEM((CHUNK,), jnp.int32),
            pltpu.VMEM((CHUNK, 128), jnp.float32),
            pltpu.VMEM((CHUNK,), jnp.int32),
            pltpu.VMEM((CHUNK,), jnp.int32),
            pltpu.VMEM((CHUNK, 128), jnp.float32),
            pltpu.VMEM_SHARED((N, 128), jnp.float32),
            pltpu.SemaphoreType.DMA,
            pltpu.SemaphoreType.DMA,
        ],
    )
    def prop(*refs):
        ts = refs[:nparts]
        zeros_hbm = refs[nparts]
        rows_hbm = refs[nparts + 1]
        cols_hbm = refs[nparts + 2]
        qs = refs[nparts + 3:nparts + 3 + nparts]
        (rowb0, colb0, gb0, rowb1, colb1, gb1,
         acc, sem0, sem1) = refs[nparts + 3 + nparts:]
        cid = lax.axis_index("c")
        sid = lax.axis_index("s")
        base = cid * (E // NC) + sid * EPT
        banks = ((rowb0, colb0, gb0, sem0), (rowb1, colb1, gb1, sem1))

        for t_hbm, q_hbm in zip(ts, qs):
            @pl.when(cid == 0)
            def _():
                _rows_copy(sid, t_hbm, acc)

            @pl.when(cid == 1)
            def _():
                _rows_copy(sid, zeros_hbm, acc)

            plsc.subcore_barrier()

            def start(j, bank):
                rowb, colb, gb, sem = bank
                off = base + j * CHUNK
                pltpu.sync_copy(rows_hbm.at[pl.ds(off, CHUNK)], rowb)
                pltpu.sync_copy(cols_hbm.at[pl.ds(off, CHUNK)], colb)
                pltpu.async_copy(t_hbm.at[rowb], gb, sem)

            def drain(bank):
                rowb, colb, gb, sem = bank
                pltpu.make_async_copy(t_hbm.at[rowb], gb, sem).wait()
                pltpu.sync_copy(gb, acc.at[colb], add=True)

            start(0, banks[0])

            def body(jj, carry):
                k0 = 2 * jj
                start(k0 + 1, banks[1])
                drain(banks[0])
                start(k0 + 2, banks[0])
                drain(banks[1])
                return carry

            lax.fori_loop(0, (nch - 1) // 2, body, 0)
            drain(banks[0])

            plsc.subcore_barrier()
            _rows_copy(sid, acc, q_hbm, doff=cid * N)

    return prop


_prop1 = _make_prop(1)
_prop3 = _make_prop(3)


@functools.partial(
    pl.kernel,
    out_type=jax.ShapeDtypeStruct((2 * N, 128), jnp.float32),
    mesh=_mesh,
    scratch_types=[
        pltpu.VMEM((CHUNK, 128), jnp.float32),
        pltpu.VMEM((CHUNK,), jnp.int32),
        pltpu.VMEM((CHUNK,), jnp.int32),
        pltpu.VMEM_SHARED((N, 128), jnp.float32),
    ],
)
def _deg_kernel(ones_c_hbm, ones_n_hbm, zeros_hbm, cols_hbm, q_hbm,
                obuf, colb0, colb1, acc):
    cid = lax.axis_index("c")
    sid = lax.axis_index("s")
    base = cid * (E // NC) + sid * EPT

    @pl.when(cid == 0)
    def _():
        _rows_copy(sid, ones_n_hbm, acc)

    @pl.when(cid == 1)
    def _():
        _rows_copy(sid, zeros_hbm, acc)

    pltpu.sync_copy(ones_c_hbm, obuf)
    plsc.subcore_barrier()

    def start(j, colb):
        pltpu.sync_copy(cols_hbm.at[pl.ds(base + j * CHUNK, CHUNK)], colb)

    start(0, colb0)

    def body(jj, carry):
        k0 = 2 * jj
        start(k0 + 1, colb1)
        pltpu.sync_copy(obuf, acc.at[colb0], add=True)
        start(k0 + 2, colb0)
        pltpu.sync_copy(obuf, acc.at[colb1], add=True)
        return carry

    nch = EPT // CHUNK
    lax.fori_loop(0, (nch - 1) // 2, body, 0)
    pltpu.sync_copy(obuf, acc.at[colb0], add=True)
    plsc.subcore_barrier()
    _rows_copy(sid, acc, q_hbm, doff=cid * N)


def _row_spec(cols):
    return pl.BlockSpec((BLK, cols), lambda i: (i, 0))


def _full_spec(r, c):
    return pl.BlockSpec((r, c), lambda i: (0, 0))


def _prep_body(qa_ref, qb_ref, x_ref, deg_ref, t1_ref):
    deg = qa_ref[...] + qb_ref[...]
    deg_ref[...] = deg[:, :16]
    t1_ref[...] = x_ref[...] * lax.rsqrt(deg[:, :1])


def _prep_call(qa, qb, x):
    return pl.pallas_call(
        _prep_body,
        grid=(NBLK,),
        in_specs=[_row_spec(128), _row_spec(128), _row_spec(128)],
        out_specs=[_row_spec(16), _row_spec(128)],
        out_shape=[jax.ShapeDtypeStruct((N, 16), jnp.float32),
                   jax.ShapeDtypeStruct((N, 128), jnp.float32)],
    )(qa, qb, x)


def _make_scale_body(nparts):
    def body(*refs):
        deg_ref = refs[2 * nparts]
        p_refs = refs[2 * nparts + 1:2 * nparts + 1 + nparts]
        t_refs = refs[2 * nparts + 1 + nparts:]
        r = 1.0 / deg_ref[:, :1]
        for k in range(nparts):
            psum = refs[2 * k][...] + refs[2 * k + 1][...]
            p_refs[k][...] = psum
            t_refs[k][...] = psum * r
    return body


def _scale_call(qparts, deg):
    nparts = len(qparts)
    flat = [a for pair in qparts for a in pair]
    shp = jax.ShapeDtypeStruct((N, 128), jnp.float32)
    out = pl.pallas_call(
        _make_scale_body(nparts),
        grid=(NBLK,),
        in_specs=[_row_spec(128)] * (2 * nparts) + [_row_spec(16)],
        out_specs=[_row_spec(128)] * (2 * nparts),
        out_shape=[shp] * (2 * nparts),
    )(*flat, deg)
    return out[:nparts], out[nparts:]


def _make_mm_body(first, nparts):
    def body(*refs):
        nh = 1 if first else nparts
        h0p = refs[:nh]
        p1p = refs[nh:nh + nparts]
        q2p = refs[nh + nparts:nh + nparts + 2 * nparts]
        deg_ref = refs[nh + 3 * nparts]
        w0_ref, w1_ref, w2_ref, b_ref = refs[nh + 3 * nparts + 1:
                                             nh + 3 * nparts + 5]
        m_ref, stats_ref = refs[nh + 3 * nparts + 5:]
        i = pl.program_id(0)
        deg = deg_ref[:, :1]
        dis = lax.rsqrt(deg)
        if first:
            h0 = h0p[0][...]
        else:
            h0 = jnp.concatenate([r[...] for r in h0p], axis=1) * jnp.sqrt(deg)
        p1 = jnp.concatenate([r[...] for r in p1p], axis=1) * dis
        p2 = jnp.concatenate(
            [q2p[2 * k][...] + q2p[2 * k + 1][...] for k in range(nparts)],
            axis=1) * dis
        dn = (((1,), (1,)), ((), ()))
        m0 = lax.dot_general(h0, w0_ref[...], dn,
                             preferred_element_type=jnp.float32)
        m1 = lax.dot_general(p1, w1_ref[...], dn,
                             preferred_element_type=jnp.float32)
        m2 = lax.dot_general(p2, w2_ref[...], dn,
                             preferred_element_type=jnp.float32)
        m = jnp.concatenate([m0, m1, m2], axis=1) + b_ref[...]
        m_ref[...] = m

        @pl.when(i == 0)
        def _():
            stats_ref[...] = jnp.zeros_like(stats_ref)

        stats_ref[0:1, :] += jnp.sum(m, axis=0, keepdims=True)
        stats_ref[1:2, :] += jnp.sum(m * m, axis=0, keepdims=True)

    return body


def _mm_call(first, h0parts, p1parts, q2parts, deg, w0, w1, w2, b):
    nparts = len(p1parts)
    din = w0.shape[1]
    q2flat = [a for pair in q2parts for a in pair]
    nin = len(h0parts) + nparts + 2 * nparts
    return pl.pallas_call(
        _make_mm_body(first, nparts),
        grid=(NBLK,),
        in_specs=[_row_spec(128)] * nin + [_row_spec(16)] +
                 [_full_spec(H, din)] * 3 + [_full_spec(1, 3 * H)],
        out_specs=[_row_spec(3 * H), _full_spec(8, 3 * H)],
        out_shape=[jax.ShapeDtypeStruct((N, 3 * H), jnp.float32),
                   jax.ShapeDtypeStruct((8, 3 * H), jnp.float32)],
    )(*h0parts, *p1parts, *q2flat, deg, w0, w1, w2, b.reshape(1, -1))


def _bn_body(m_ref, stats_ref, deg_ref, g_ref, be_ref, t0_ref, t1_ref, t2_ref):
    mu = stats_ref[0:1, :] * (1.0 / N)
    var = stats_ref[1:2, :] * (1.0 / N) - mu * mu
    y = (m_ref[...] - mu) * lax.rsqrt(var + EPS) * g_ref[...] + be_ref[...]
    y = jnp.where(y >= 0.0, y, 0.1 * y)
    t = y * lax.rsqrt(deg_ref[:, :1])
    t0_ref[...] = t[:, 0:128]
    t1_ref[...] = t[:, 128:256]
    t2_ref[...] = t[:, 256:384]


def _bn_call(m, stats, deg, g, be):
    shp = jax.ShapeDtypeStruct((N, 128), jnp.float32)
    return pl.pallas_call(
        _bn_body,
        grid=(NBLK,),
        in_specs=[_row_spec(3 * H), _full_spec(8, 3 * H), _row_spec(16),
                  _full_spec(1, 3 * H), _full_spec(1, 3 * H)],
        out_specs=[_row_spec(128)] * 3,
        out_shape=[shp] * 3,
    )(m, stats, deg, g.reshape(1, -1), be.reshape(1, -1))


def _safe_pow(x, p):
    safe = jnp.where(x > 0.0, x, 1.0)
    return jnp.where(x > 0.0, jnp.exp(p * jnp.log(safe)), 0.0)


def _final_body(m_ref, stats_ref, g_ref, be_ref, p_ref, wg_ref, bg_ref,
                logits_ref, ypred_ref, psum_ref):
    i = pl.program_id(0)
    mu = stats_ref[0:1, :] * (1.0 / N)
    var = stats_ref[1:2, :] * (1.0 / N) - mu * mu
    y = (m_ref[...] - mu) * lax.rsqrt(var + EPS) * g_ref[...] + be_ref[...]
    y = jnp.where(y >= 0.0, y, 0.1 * y)
    pp = p_ref[0, 0]
    s = _safe_pow(jnp.clip(y, 0.0, 100.0), pp)

    @pl.when(i == 0)
    def _():
        psum_ref[...] = jnp.zeros_like(psum_ref)

    psum_ref[0:1, :] += jnp.sum(s, axis=0, keepdims=True)

    @pl.when(i == NBLK - 1)
    def _():
        mean = psum_ref[0:1, :] * (1.0 / N)
        pooled = _safe_pow(jnp.clip(mean, 0.0, 100.0), 1.0 / pp)
        dn = (((1,), (1,)), ((), ()))
        logits = lax.dot_general(pooled, wg_ref[...], dn,
                                 preferred_element_type=jnp.float32)
        logits = logits + bg_ref[...]
        logits_ref[...] = logits
        iota = lax.broadcasted_iota(jnp.int32, (1, 10), 1)
        mx = jnp.max(logits, axis=1, keepdims=True)
        ypred_ref[...] = jnp.min(jnp.where(logits == mx, iota, 10),
                                 axis=1, keepdims=True)


def _final_call(m, stats, g, be, p, wg, bg):
    return pl.pallas_call(
        _final_body,
        grid=(NBLK,),
        in_specs=[_row_spec(3 * H), _full_spec(8, 3 * H),
                  _full_spec(1, 3 * H), _full_spec(1, 3 * H),
                  _full_spec(1, 1), _full_spec(10, 3 * EMB), _full_spec(1, 10)],
        out_specs=[_full_spec(1, 10), _full_spec(1, 1)],
        out_shape=[jax.ShapeDtypeStruct((1, 10), jnp.float32),
                   jax.ShapeDtypeStruct((1, 1), jnp.int32)],
        scratch_shapes=[pltpu.VMEM((8, 3 * H), jnp.float32)],
    )(m, stats, g.reshape(1, -1), be.reshape(1, -1), p.reshape(1, 1),
      wg, bg.reshape(1, -1))


def _halves(q):
    return (q[:N], q[N:])


def _run_prop(propfn, ts, zeros_n, rows, cols):
    out = propfn(*ts, zeros_n, rows, cols)
    return list(out) if isinstance(out, (list, tuple)) else [out]


def kernel(x, edge_index, Wf0, Wf1, Wf2, bf, Wb0, Wb1, Wb2, bb,
           Wl0, Wl1, Wl2, bl, g1, be1, g2, be2, g3, be3, p, Wg, bg):
    rows = edge_index[0]
    cols = edge_index[1]
    ones_c = jnp.ones((CHUNK, 128), jnp.float32)
    ones_n = jnp.ones((N, 128), jnp.float32)
    zeros_n = jnp.zeros((N, 128), jnp.float32)

    qdeg = _deg_kernel(ones_c, ones_n, zeros_n, cols)
    deg, t1 = _prep_call(qdeg[:N], qdeg[N:], x)

    [q1] = _run_prop(_prop1, [t1], zeros_n, rows, cols)
    (p1,), (t2,) = _scale_call([_halves(q1)], deg)
    [q2] = _run_prop(_prop1, [t2], zeros_n, rows, cols)
    m1, s1 = _mm_call(True, [x], [p1], [_halves(q2)], deg, Wf0, Wf1, Wf2, bf)
    t3 = _bn_call(m1, s1, deg, g1, be1)

    q3 = _run_prop(_prop3, t3, zeros_n, rows, cols)
    p3, t4 = _scale_call([_halves(q) for q in q3], deg)
    q4 = _run_prop(_prop3, t4, zeros_n, rows, cols)
    m2, s2 = _mm_call(False, t3, p3, [_halves(q) for q in q4], deg,
                      Wb0, Wb1, Wb2, bb)
    t5 = _bn_call(m2, s2, deg, g2, be2)

    q5 = _run_prop(_prop3, t5, zeros_n, rows, cols)
    p5, t6 = _scale_call([_halves(q) for q in q5], deg)
    q6 = _run_prop(_prop3, t6, zeros_n, rows, cols)
    m3, s3 = _mm_call(False, t5, p5, [_halves(q) for q in q6], deg,
                      Wl0, Wl1, Wl2, bl)

    logits, ypred = _final_call(m3, s3, g3, be3, p, Wg, bg)
    return (logits, ypred.reshape(-1))


# speedup vs baseline: 14.4569x; 1.5697x over previous
import functools

import jax
import jax.numpy as jnp
from jax import lax
from jax.experimental import pallas as pl
from jax.experimental.pallas import tpu as pltpu
from jax.experimental.pallas import tpu_sc as plsc

N = 10000
E = 320000
H = 128
EMB = 128
EPS = 1e-5
NS = 16
NC = 2
RPT = 624
TAIL0 = NS * RPT
TAIL = N - TAIL0
CHUNK = 80
GCH = 128
EPT = E // (NC * NS)
NCHF = EPT // GCH
TAILE = EPT - NCHF * GCH
BLK = 2000
NBLK = N // BLK

_mesh = plsc.VectorSubcoreMesh(core_axis_name="c", subcore_axis_name="s")


def _rows_copy(sid, src, dst, soff=0, doff=0):
    b = sid * RPT
    pltpu.sync_copy(src.at[pl.ds(soff + b, RPT)], dst.at[pl.ds(doff + b, RPT)])

    @pl.when(sid == 0)
    def _():
        pltpu.sync_copy(src.at[pl.ds(soff + TAIL0, TAIL)],
                        dst.at[pl.ds(doff + TAIL0, TAIL)])


def _make_prop(nparts):
    @functools.partial(
        pl.kernel,
        out_type=[jax.ShapeDtypeStruct((2 * N, 128), jnp.float32)
                  for _ in range(nparts)],
        mesh=_mesh,
        scratch_types=[
            pltpu.VMEM((EPT,), jnp.int32),
            pltpu.VMEM((GCH,), jnp.int32),
            pltpu.VMEM((GCH, 128), jnp.float32),
            pltpu.VMEM((GCH,), jnp.int32),
            pltpu.VMEM((GCH, 128), jnp.float32),
            pltpu.VMEM((TAILE,), jnp.int32),
            pltpu.VMEM((TAILE, 128), jnp.float32),
            pltpu.VMEM_SHARED((N, 128), jnp.float32),
            pltpu.SemaphoreType.DMA,
            pltpu.SemaphoreType.DMA,
            pltpu.SemaphoreType.DMA,
            pltpu.SemaphoreType.DMA,
            pltpu.SemaphoreType.DMA,
        ],
    )
    def prop(*refs):
        ts = refs[:nparts]
        zeros_hbm = refs[nparts]
        rows_hbm = refs[nparts + 1]
        cols_hbm = refs[nparts + 2]
        qs = refs[nparts + 3:nparts + 3 + nparts]
        (rows_all, colb0, gb0, colb1, gb1, colbt, gbt,
         acc, sem0, sem1, semt, semi0, semi1) = refs[nparts + 3 + nparts:]
        cid = lax.axis_index("c")
        sid = lax.axis_index("s")
        base = cid * (E // NC) + sid * EPT
        banks = ((colb0, gb0, sem0, semi0), (colb1, gb1, sem1, semi1))

        pltpu.sync_copy(rows_hbm.at[pl.ds(base, EPT)], rows_all)

        for t_hbm, q_hbm in zip(ts, qs):
            @pl.when(cid == 0)
            def _():
                _rows_copy(sid, t_hbm, acc)

            @pl.when(cid == 1)
            def _():
                _rows_copy(sid, zeros_hbm, acc)

            plsc.subcore_barrier()

            def start(j, bank):
                colb, gb, sem, semi = bank
                pltpu.async_copy(
                    cols_hbm.at[pl.ds(base + j * GCH, GCH)], colb, semi)
                pltpu.async_copy(
                    t_hbm.at[rows_all.at[pl.ds(j * GCH, GCH)]], gb, sem)

            def drain(bank):
                colb, gb, sem, semi = bank
                pltpu.make_async_copy(
                    cols_hbm.at[pl.ds(base, GCH)], colb, semi).wait()
                pltpu.make_async_copy(
                    t_hbm.at[rows_all.at[pl.ds(0, GCH)]], gb, sem).wait()
                pltpu.sync_copy(gb, acc.at[colb], add=True)

            start(0, banks[0])

            def body(jj, carry):
                k0 = 2 * jj
                start(k0 + 1, banks[1])
                drain(banks[0])
                start(k0 + 2, banks[0])
                drain(banks[1])
                return carry

            lax.fori_loop(0, (NCHF - 1) // 2, body, 0)
            if NCHF % 2 == 0:
                start(NCHF - 1, banks[1])
                drain(banks[0])
                drain(banks[1])
            else:
                drain(banks[0])
            if TAILE:
                off = base + NCHF * GCH
                pltpu.sync_copy(cols_hbm.at[pl.ds(off, TAILE)], colbt)
                pltpu.async_copy(
                    t_hbm.at[rows_all.at[pl.ds(NCHF * GCH, TAILE)]],
                    gbt, semt).wait()
                pltpu.sync_copy(gbt, acc.at[colbt], add=True)

            plsc.subcore_barrier()
            _rows_copy(sid, acc, q_hbm, doff=cid * N)

    return prop


_prop1 = _make_prop(1)
_prop3 = _make_prop(3)


@functools.partial(
    pl.kernel,
    out_type=jax.ShapeDtypeStruct((2 * N, 128), jnp.float32),
    mesh=_mesh,
    scratch_types=[
        pltpu.VMEM((CHUNK, 128), jnp.float32),
        pltpu.VMEM((CHUNK,), jnp.int32),
        pltpu.VMEM((CHUNK,), jnp.int32),
        pltpu.VMEM_SHARED((N, 128), jnp.float32),
    ],
)
def _deg_kernel(ones_c_hbm, ones_n_hbm, zeros_hbm, cols_hbm, q_hbm,
                obuf, colb0, colb1, acc):
    cid = lax.axis_index("c")
    sid = lax.axis_index("s")
    base = cid * (E // NC) + sid * EPT

    @pl.when(cid == 0)
    def _():
        _rows_copy(sid, ones_n_hbm, acc)

    @pl.when(cid == 1)
    def _():
        _rows_copy(sid, zeros_hbm, acc)

    pltpu.sync_copy(ones_c_hbm, obuf)
    plsc.subcore_barrier()

    def start(j, colb):
        pltpu.sync_copy(cols_hbm.at[pl.ds(base + j * CHUNK, CHUNK)], colb)

    start(0, colb0)

    def body(jj, carry):
        k0 = 2 * jj
        start(k0 + 1, colb1)
        pltpu.sync_copy(obuf, acc.at[colb0], add=True)
        start(k0 + 2, colb0)
        pltpu.sync_copy(obuf, acc.at[colb1], add=True)
        return carry

    nch = EPT // CHUNK
    lax.fori_loop(0, (nch - 1) // 2, body, 0)
    pltpu.sync_copy(obuf, acc.at[colb0], add=True)
    plsc.subcore_barrier()
    _rows_copy(sid, acc, q_hbm, doff=cid * N)


def _row_spec(cols):
    return pl.BlockSpec((BLK, cols), lambda i: (i, 0))


def _full_spec(r, c):
    return pl.BlockSpec((r, c), lambda i: (0, 0))


def _prep_body(qa_ref, qb_ref, x_ref, deg_ref, t1_ref):
    deg = qa_ref[...] + qb_ref[...]
    deg_ref[...] = deg[:, :16]
    t1_ref[...] = x_ref[...] * lax.rsqrt(deg[:, :1])


def _prep_call(qa, qb, x):
    return pl.pallas_call(
        _prep_body,
        grid=(NBLK,),
        in_specs=[_row_spec(128), _row_spec(128), _row_spec(128)],
        out_specs=[_row_spec(16), _row_spec(128)],
        out_shape=[jax.ShapeDtypeStruct((N, 16), jnp.float32),
                   jax.ShapeDtypeStruct((N, 128), jnp.float32)],
    )(qa, qb, x)


def _make_scale_body(nparts):
    def body(*refs):
        deg_ref = refs[2 * nparts]
        p_refs = refs[2 * nparts + 1:2 * nparts + 1 + nparts]
        t_refs = refs[2 * nparts + 1 + nparts:]
        r = 1.0 / deg_ref[:, :1]
        for k in range(nparts):
            psum = refs[2 * k][...] + refs[2 * k + 1][...]
            p_refs[k][...] = psum
            t_refs[k][...] = psum * r
    return body


def _scale_call(qparts, deg):
    nparts = len(qparts)
    flat = [a for pair in qparts for a in pair]
    shp = jax.ShapeDtypeStruct((N, 128), jnp.float32)
    out = pl.pallas_call(
        _make_scale_body(nparts),
        grid=(NBLK,),
        in_specs=[_row_spec(128)] * (2 * nparts) + [_row_spec(16)],
        out_specs=[_row_spec(128)] * (2 * nparts),
        out_shape=[shp] * (2 * nparts),
    )(*flat, deg)
    return out[:nparts], out[nparts:]


def _make_mm_body(first, nparts):
    def body(*refs):
        nh = 1 if first else nparts
        h0p = refs[:nh]
        p1p = refs[nh:nh + nparts]
        q2p = refs[nh + nparts:nh + nparts + 2 * nparts]
        deg_ref = refs[nh + 3 * nparts]
        w0_ref, w1_ref, w2_ref, b_ref = refs[nh + 3 * nparts + 1:
                                             nh + 3 * nparts + 5]
        m_ref, stats_ref = refs[nh + 3 * nparts + 5:]
        i = pl.program_id(0)
        deg = deg_ref[:, :1]
        dis = lax.rsqrt(deg)
        if first:
            h0 = h0p[0][...]
        else:
            h0 = jnp.concatenate([r[...] for r in h0p], axis=1) * jnp.sqrt(deg)
        p1 = jnp.concatenate([r[...] for r in p1p], axis=1) * dis
        p2 = jnp.concatenate(
            [q2p[2 * k][...] + q2p[2 * k + 1][...] for k in range(nparts)],
            axis=1) * dis
        dn = (((1,), (1,)), ((), ()))
        m0 = lax.dot_general(h0, w0_ref[...], dn,
                             preferred_element_type=jnp.float32)
        m1 = lax.dot_general(p1, w1_ref[...], dn,
                             preferred_element_type=jnp.float32)
        m2 = lax.dot_general(p2, w2_ref[...], dn,
                             preferred_element_type=jnp.float32)
        m = jnp.concatenate([m0, m1, m2], axis=1) + b_ref[...]
        m_ref[...] = m

        @pl.when(i == 0)
        def _():
            stats_ref[...] = jnp.zeros_like(stats_ref)

        stats_ref[0:1, :] += jnp.sum(m, axis=0, keepdims=True)
        stats_ref[1:2, :] += jnp.sum(m * m, axis=0, keepdims=True)

    return body


def _mm_call(first, h0parts, p1parts, q2parts, deg, w0, w1, w2, b):
    nparts = len(p1parts)
    din = w0.shape[1]
    q2flat = [a for pair in q2parts for a in pair]
    nin = len(h0parts) + nparts + 2 * nparts
    return pl.pallas_call(
        _make_mm_body(first, nparts),
        grid=(NBLK,),
        in_specs=[_row_spec(128)] * nin + [_row_spec(16)] +
                 [_full_spec(H, din)] * 3 + [_full_spec(1, 3 * H)],
        out_specs=[_row_spec(3 * H), _full_spec(8, 3 * H)],
        out_shape=[jax.ShapeDtypeStruct((N, 3 * H), jnp.float32),
                   jax.ShapeDtypeStruct((8, 3 * H), jnp.float32)],
    )(*h0parts, *p1parts, *q2flat, deg, w0, w1, w2, b.reshape(1, -1))


def _bn_body(m_ref, stats_ref, deg_ref, g_ref, be_ref, t0_ref, t1_ref, t2_ref):
    mu = stats_ref[0:1, :] * (1.0 / N)
    var = stats_ref[1:2, :] * (1.0 / N) - mu * mu
    y = (m_ref[...] - mu) * lax.rsqrt(var + EPS) * g_ref[...] + be_ref[...]
    y = jnp.where(y >= 0.0, y, 0.1 * y)
    t = y * lax.rsqrt(deg_ref[:, :1])
    t0_ref[...] = t[:, 0:128]
    t1_ref[...] = t[:, 128:256]
    t2_ref[...] = t[:, 256:384]


def _bn_call(m, stats, deg, g, be):
    shp = jax.ShapeDtypeStruct((N, 128), jnp.float32)
    return pl.pallas_call(
        _bn_body,
        grid=(NBLK,),
        in_specs=[_row_spec(3 * H), _full_spec(8, 3 * H), _row_spec(16),
                  _full_spec(1, 3 * H), _full_spec(1, 3 * H)],
        out_specs=[_row_spec(128)] * 3,
        out_shape=[shp] * 3,
    )(m, stats, deg, g.reshape(1, -1), be.reshape(1, -1))


def _safe_pow(x, p):
    safe = jnp.where(x > 0.0, x, 1.0)
    return jnp.where(x > 0.0, jnp.exp(p * jnp.log(safe)), 0.0)


def _final_body(m_ref, stats_ref, g_ref, be_ref, p_ref, wg_ref, bg_ref,
                logits_ref, ypred_ref, psum_ref):
    i = pl.program_id(0)
    mu = stats_ref[0:1, :] * (1.0 / N)
    var = stats_ref[1:2, :] * (1.0 / N) - mu * mu
    y = (m_ref[...] - mu) * lax.rsqrt(var + EPS) * g_ref[...] + be_ref[...]
    y = jnp.where(y >= 0.0, y, 0.1 * y)
    pp = p_ref[0, 0]
    s = _safe_pow(jnp.clip(y, 0.0, 100.0), pp)

    @pl.when(i == 0)
    def _():
        psum_ref[...] = jnp.zeros_like(psum_ref)

    psum_ref[0:1, :] += jnp.sum(s, axis=0, keepdims=True)

    @pl.when(i == NBLK - 1)
    def _():
        mean = psum_ref[0:1, :] * (1.0 / N)
        pooled = _safe_pow(jnp.clip(mean, 0.0, 100.0), 1.0 / pp)
        dn = (((1,), (1,)), ((), ()))
        logits = lax.dot_general(pooled, wg_ref[...], dn,
                                 preferred_element_type=jnp.float32)
        logits = logits + bg_ref[...]
        logits_ref[...] = logits
        iota = lax.broadcasted_iota(jnp.int32, (1, 10), 1)
        mx = jnp.max(logits, axis=1, keepdims=True)
        ypred_ref[...] = jnp.min(jnp.where(logits == mx, iota, 10),
                                 axis=1, keepdims=True)


def _final_call(m, stats, g, be, p, wg, bg):
    return pl.pallas_call(
        _final_body,
        grid=(NBLK,),
        in_specs=[_row_spec(3 * H), _full_spec(8, 3 * H),
                  _full_spec(1, 3 * H), _full_spec(1, 3 * H),
                  _full_spec(1, 1), _full_spec(10, 3 * EMB), _full_spec(1, 10)],
        out_specs=[_full_spec(1, 10), _full_spec(1, 1)],
        out_shape=[jax.ShapeDtypeStruct((1, 10), jnp.float32),
                   jax.ShapeDtypeStruct((1, 1), jnp.int32)],
        scratch_shapes=[pltpu.VMEM((8, 3 * H), jnp.float32)],
    )(m, stats, g.reshape(1, -1), be.reshape(1, -1), p.reshape(1, 1),
      wg, bg.reshape(1, -1))


def _halves(q):
    return (q[:N], q[N:])


def _run_prop(propfn, ts, zeros_n, rows, cols):
    out = propfn(*ts, zeros_n, rows, cols)
    return list(out) if isinstance(out, (list, tuple)) else [out]


def kernel(x, edge_index, Wf0, Wf1, Wf2, bf, Wb0, Wb1, Wb2, bb,
           Wl0, Wl1, Wl2, bl, g1, be1, g2, be2, g3, be3, p, Wg, bg):
    rows = edge_index[0]
    cols = edge_index[1]
    ones_c = jnp.ones((CHUNK, 128), jnp.float32)
    ones_n = jnp.ones((N, 128), jnp.float32)
    zeros_n = jnp.zeros((N, 128), jnp.float32)

    qdeg = _deg_kernel(ones_c, ones_n, zeros_n, cols)
    deg, t1 = _prep_call(qdeg[:N], qdeg[N:], x)

    [q1] = _run_prop(_prop1, [t1], zeros_n, rows, cols)
    (p1,), (t2,) = _scale_call([_halves(q1)], deg)
    [q2] = _run_prop(_prop1, [t2], zeros_n, rows, cols)
    m1, s1 = _mm_call(True, [x], [p1], [_halves(q2)], deg, Wf0, Wf1, Wf2, bf)
    t3 = _bn_call(m1, s1, deg, g1, be1)

    q3 = _run_prop(_prop3, t3, zeros_n, rows, cols)
    p3, t4 = _scale_call([_halves(q) for q in q3], deg)
    q4 = _run_prop(_prop3, t4, zeros_n, rows, cols)
    m2, s2 = _mm_call(False, t3, p3, [_halves(q) for q in q4], deg,
                      Wb0, Wb1, Wb2, bb)
    t5 = _bn_call(m2, s2, deg, g2, be2)

    q5 = _run_prop(_prop3, t5, zeros_n, rows, cols)
    p5, t6 = _scale_call([_halves(q) for q in q5], deg)
    q6 = _run_prop(_prop3, t6, zeros_n, rows, cols)
    m3, s3 = _mm_call(False, t5, p5, [_halves(q) for q in q6], deg,
                      Wl0, Wl1, Wl2, bl)

    logits, ypred = _final_call(m3, s3, g3, be3, p, Wg, bg)
    return (logits, ypred.reshape(-1))


# speedup vs baseline: 15.2374x; 1.0540x over previous
import functools

import jax
import jax.numpy as jnp
from jax import lax
from jax.experimental import pallas as pl
from jax.experimental.pallas import tpu as pltpu
from jax.experimental.pallas import tpu_sc as plsc

N = 10000
E = 320000
H = 128
EMB = 128
EPS = 1e-5
NS = 16
NC = 2
RPT = 624
TAIL0 = NS * RPT
TAIL = N - TAIL0
CHUNK = 80
GCH = 80
EPT = E // (NC * NS)
NCHF = EPT // GCH
BLK = 2000
NBLK = N // BLK

_mesh = plsc.VectorSubcoreMesh(core_axis_name="c", subcore_axis_name="s")


def _rows_copy(sid, src, dst, soff=0, doff=0):
    b = sid * RPT
    pltpu.sync_copy(src.at[pl.ds(soff + b, RPT)], dst.at[pl.ds(doff + b, RPT)])

    @pl.when(sid == 0)
    def _():
        pltpu.sync_copy(src.at[pl.ds(soff + TAIL0, TAIL)],
                        dst.at[pl.ds(doff + TAIL0, TAIL)])


def _make_prop(nparts):
    @functools.partial(
        pl.kernel,
        out_type=[jax.ShapeDtypeStruct((2 * N, 128), jnp.float32)
                  for _ in range(nparts)],
        mesh=_mesh,
        scratch_types=(
            [pltpu.VMEM((EPT,), jnp.int32)] +
            [pltpu.VMEM((GCH,), jnp.int32)] * 3 +
            [pltpu.VMEM((GCH, 128), jnp.float32)] * 3 +
            [pltpu.VMEM_SHARED((N, 128), jnp.float32)] +
            [pltpu.SemaphoreType.DMA] * 9
        ),
    )
    def prop(*refs):
        ts = refs[:nparts]
        zeros_hbm = refs[nparts]
        rows_hbm = refs[nparts + 1]
        cols_hbm = refs[nparts + 2]
        qs = refs[nparts + 3:nparts + 3 + nparts]
        sc = refs[nparts + 3 + nparts:]
        rows_all = sc[0]
        colbs = sc[1:4]
        gbs = sc[4:7]
        acc = sc[7]
        semgs = sc[8:11]
        semis = sc[11:14]
        semss = sc[14:17]
        cid = lax.axis_index("c")
        sid = lax.axis_index("s")
        base = cid * (E // NC) + sid * EPT
        banks = tuple(
            (colbs[b], gbs[b], semgs[b], semis[b], semss[b]) for b in range(3))

        pltpu.sync_copy(rows_hbm.at[pl.ds(base, EPT)], rows_all)

        for t_hbm, q_hbm in zip(ts, qs):
            @pl.when(cid == 0)
            def _():
                _rows_copy(sid, t_hbm, acc)

            @pl.when(cid == 1)
            def _():
                _rows_copy(sid, zeros_hbm, acc)

            plsc.subcore_barrier()

            def start(j, bank):
                colb, gb, semg, semi, sems = bank
                pltpu.async_copy(
                    cols_hbm.at[pl.ds(base + j * GCH, GCH)], colb, semi)
                pltpu.async_copy(
                    t_hbm.at[rows_all.at[pl.ds(j * GCH, GCH)]], gb, semg)

            def scat(j, bank):
                colb, gb, semg, semi, sems = bank
                pltpu.make_async_copy(
                    cols_hbm.at[pl.ds(base, GCH)], colb, semi).wait()
                pltpu.make_async_copy(
                    t_hbm.at[rows_all.at[pl.ds(0, GCH)]], gb, semg).wait()
                pltpu.async_copy(gb, acc.at[colb], sems, add=True)

            def waits(bank):
                colb, gb, semg, semi, sems = bank
                pltpu.make_async_copy(gb, acc.at[colb], sems).wait()

            start(0, banks[0])
            start(1, banks[1])
            scat(0, banks[0])
            start(2, banks[2])

            def body(jj, carry):
                k0 = 3 * jj + 1
                for o in range(3):
                    k = k0 + o
                    scat(k, banks[(1 + o) % 3])
                    waits(banks[o % 3])
                    start(k + 2, banks[o % 3])
                return carry

            lax.fori_loop(0, 40, body, 0)
            scat(121, banks[1])
            waits(banks[0])
            start(123, banks[0])
            scat(122, banks[2])
            waits(banks[1])
            start(124, banks[1])
            scat(123, banks[0])
            waits(banks[2])
            scat(124, banks[1])
            waits(banks[0])
            waits(banks[1])

            plsc.subcore_barrier()
            _rows_copy(sid, acc, q_hbm, doff=cid * N)

    return prop


_prop1 = _make_prop(1)
_prop3 = _make_prop(3)


@functools.partial(
    pl.kernel,
    out_type=jax.ShapeDtypeStruct((2 * N, 128), jnp.float32),
    mesh=_mesh,
    scratch_types=[
        pltpu.VMEM((CHUNK, 128), jnp.float32),
        pltpu.VMEM((CHUNK,), jnp.int32),
        pltpu.VMEM((CHUNK,), jnp.int32),
        pltpu.VMEM_SHARED((N, 128), jnp.float32),
    ],
)
def _deg_kernel(ones_c_hbm, ones_n_hbm, zeros_hbm, cols_hbm, q_hbm,
                obuf, colb0, colb1, acc):
    cid = lax.axis_index("c")
    sid = lax.axis_index("s")
    base = cid * (E // NC) + sid * EPT

    @pl.when(cid == 0)
    def _():
        _rows_copy(sid, ones_n_hbm, acc)

    @pl.when(cid == 1)
    def _():
        _rows_copy(sid, zeros_hbm, acc)

    pltpu.sync_copy(ones_c_hbm, obuf)
    plsc.subcore_barrier()

    def start(j, colb):
        pltpu.sync_copy(cols_hbm.at[pl.ds(base + j * CHUNK, CHUNK)], colb)

    start(0, colb0)

    def body(jj, carry):
        k0 = 2 * jj
        start(k0 + 1, colb1)
        pltpu.sync_copy(obuf, acc.at[colb0], add=True)
        start(k0 + 2, colb0)
        pltpu.sync_copy(obuf, acc.at[colb1], add=True)
        return carry

    nch = EPT // CHUNK
    lax.fori_loop(0, (nch - 1) // 2, body, 0)
    pltpu.sync_copy(obuf, acc.at[colb0], add=True)
    plsc.subcore_barrier()
    _rows_copy(sid, acc, q_hbm, doff=cid * N)


def _row_spec(cols):
    return pl.BlockSpec((BLK, cols), lambda i: (i, 0))


def _full_spec(r, c):
    return pl.BlockSpec((r, c), lambda i: (0, 0))


def _prep_body(qa_ref, qb_ref, x_ref, deg_ref, t1_ref):
    deg = qa_ref[...] + qb_ref[...]
    deg_ref[...] = deg[:, :16]
    t1_ref[...] = x_ref[...] * lax.rsqrt(deg[:, :1])


def _prep_call(qa, qb, x):
    return pl.pallas_call(
        _prep_body,
        grid=(NBLK,),
        in_specs=[_row_spec(128), _row_spec(128), _row_spec(128)],
        out_specs=[_row_spec(16), _row_spec(128)],
        out_shape=[jax.ShapeDtypeStruct((N, 16), jnp.float32),
                   jax.ShapeDtypeStruct((N, 128), jnp.float32)],
    )(qa, qb, x)


def _make_scale_body(nparts):
    def body(*refs):
        deg_ref = refs[2 * nparts]
        p_refs = refs[2 * nparts + 1:2 * nparts + 1 + nparts]
        t_refs = refs[2 * nparts + 1 + nparts:]
        r = 1.0 / deg_ref[:, :1]
        for k in range(nparts):
            psum = refs[2 * k][...] + refs[2 * k + 1][...]
            p_refs[k][...] = psum
            t_refs[k][...] = psum * r
    return body


def _scale_call(qparts, deg):
    nparts = len(qparts)
    flat = [a for pair in qparts for a in pair]
    shp = jax.ShapeDtypeStruct((N, 128), jnp.float32)
    out = pl.pallas_call(
        _make_scale_body(nparts),
        grid=(NBLK,),
        in_specs=[_row_spec(128)] * (2 * nparts) + [_row_spec(16)],
        out_specs=[_row_spec(128)] * (2 * nparts),
        out_shape=[shp] * (2 * nparts),
    )(*flat, deg)
    return out[:nparts], out[nparts:]


def _make_mm_body(first, nparts):
    def body(*refs):
        nh = 1 if first else nparts
        h0p = refs[:nh]
        p1p = refs[nh:nh + nparts]
        q2p = refs[nh + nparts:nh + nparts + 2 * nparts]
        deg_ref = refs[nh + 3 * nparts]
        w0_ref, w1_ref, w2_ref, b_ref = refs[nh + 3 * nparts + 1:
                                             nh + 3 * nparts + 5]
        m_ref, stats_ref = refs[nh + 3 * nparts + 5:]
        i = pl.program_id(0)
        deg = deg_ref[:, :1]
        dis = lax.rsqrt(deg)
        if first:
            h0 = h0p[0][...]
        else:
            h0 = jnp.concatenate([r[...] for r in h0p], axis=1) * jnp.sqrt(deg)
        p1 = jnp.concatenate([r[...] for r in p1p], axis=1) * dis
        p2 = jnp.concatenate(
            [q2p[2 * k][...] + q2p[2 * k + 1][...] for k in range(nparts)],
            axis=1) * dis
        dn = (((1,), (1,)), ((), ()))
        m0 = lax.dot_general(h0, w0_ref[...], dn,
                             preferred_element_type=jnp.float32)
        m1 = lax.dot_general(p1, w1_ref[...], dn,
                             preferred_element_type=jnp.float32)
        m2 = lax.dot_general(p2, w2_ref[...], dn,
                             preferred_element_type=jnp.float32)
        m = jnp.concatenate([m0, m1, m2], axis=1) + b_ref[...]
        m_ref[...] = m

        @pl.when(i == 0)
        def _():
            stats_ref[...] = jnp.zeros_like(stats_ref)

        stats_ref[0:1, :] += jnp.sum(m, axis=0, keepdims=True)
        stats_ref[1:2, :] += jnp.sum(m * m, axis=0, keepdims=True)

    return body


def _mm_call(first, h0parts, p1parts, q2parts, deg, w0, w1, w2, b):
    nparts = len(p1parts)
    din = w0.shape[1]
    q2flat = [a for pair in q2parts for a in pair]
    nin = len(h0parts) + nparts + 2 * nparts
    return pl.pallas_call(
        _make_mm_body(first, nparts),
        grid=(NBLK,),
        in_specs=[_row_spec(128)] * nin + [_row_spec(16)] +
                 [_full_spec(H, din)] * 3 + [_full_spec(1, 3 * H)],
        out_specs=[_row_spec(3 * H), _full_spec(8, 3 * H)],
        out_shape=[jax.ShapeDtypeStruct((N, 3 * H), jnp.float32),
                   jax.ShapeDtypeStruct((8, 3 * H), jnp.float32)],
    )(*h0parts, *p1parts, *q2flat, deg, w0, w1, w2, b.reshape(1, -1))


def _bn_body(m_ref, stats_ref, deg_ref, g_ref, be_ref, t0_ref, t1_ref, t2_ref):
    mu = stats_ref[0:1, :] * (1.0 / N)
    var = stats_ref[1:2, :] * (1.0 / N) - mu * mu
    y = (m_ref[...] - mu) * lax.rsqrt(var + EPS) * g_ref[...] + be_ref[...]
    y = jnp.where(y >= 0.0, y, 0.1 * y)
    t = y * lax.rsqrt(deg_ref[:, :1])
    t0_ref[...] = t[:, 0:128]
    t1_ref[...] = t[:, 128:256]
    t2_ref[...] = t[:, 256:384]


def _bn_call(m, stats, deg, g, be):
    shp = jax.ShapeDtypeStruct((N, 128), jnp.float32)
    return pl.pallas_call(
        _bn_body,
        grid=(NBLK,),
        in_specs=[_row_spec(3 * H), _full_spec(8, 3 * H), _row_spec(16),
                  _full_spec(1, 3 * H), _full_spec(1, 3 * H)],
        out_specs=[_row_spec(128)] * 3,
        out_shape=[shp] * 3,
    )(m, stats, deg, g.reshape(1, -1), be.reshape(1, -1))


def _safe_pow(x, p):
    safe = jnp.where(x > 0.0, x, 1.0)
    return jnp.where(x > 0.0, jnp.exp(p * jnp.log(safe)), 0.0)


def _final_body(m_ref, stats_ref, g_ref, be_ref, p_ref, wg_ref, bg_ref,
                logits_ref, ypred_ref, psum_ref):
    i = pl.program_id(0)
    mu = stats_ref[0:1, :] * (1.0 / N)
    var = stats_ref[1:2, :] * (1.0 / N) - mu * mu
    y = (m_ref[...] - mu) * lax.rsqrt(var + EPS) * g_ref[...] + be_ref[...]
    y = jnp.where(y >= 0.0, y, 0.1 * y)
    pp = p_ref[0, 0]
    s = _safe_pow(jnp.clip(y, 0.0, 100.0), pp)

    @pl.when(i == 0)
    def _():
        psum_ref[...] = jnp.zeros_like(psum_ref)

    psum_ref[0:1, :] += jnp.sum(s, axis=0, keepdims=True)

    @pl.when(i == NBLK - 1)
    def _():
        mean = psum_ref[0:1, :] * (1.0 / N)
        pooled = _safe_pow(jnp.clip(mean, 0.0, 100.0), 1.0 / pp)
        dn = (((1,), (1,)), ((), ()))
        logits = lax.dot_general(pooled, wg_ref[...], dn,
                                 preferred_element_type=jnp.float32)
        logits = logits + bg_ref[...]
        logits_ref[...] = logits
        iota = lax.broadcasted_iota(jnp.int32, (1, 10), 1)
        mx = jnp.max(logits, axis=1, keepdims=True)
        ypred_ref[...] = jnp.min(jnp.where(logits == mx, iota, 10),
                                 axis=1, keepdims=True)


def _final_call(m, stats, g, be, p, wg, bg):
    return pl.pallas_call(
        _final_body,
        grid=(NBLK,),
        in_specs=[_row_spec(3 * H), _full_spec(8, 3 * H),
                  _full_spec(1, 3 * H), _full_spec(1, 3 * H),
                  _full_spec(1, 1), _full_spec(10, 3 * EMB), _full_spec(1, 10)],
        out_specs=[_full_spec(1, 10), _full_spec(1, 1)],
        out_shape=[jax.ShapeDtypeStruct((1, 10), jnp.float32),
                   jax.ShapeDtypeStruct((1, 1), jnp.int32)],
        scratch_shapes=[pltpu.VMEM((8, 3 * H), jnp.float32)],
    )(m, stats, g.reshape(1, -1), be.reshape(1, -1), p.reshape(1, 1),
      wg, bg.reshape(1, -1))


def _halves(q):
    return (q[:N], q[N:])


def _run_prop(propfn, ts, zeros_n, rows, cols):
    out = propfn(*ts, zeros_n, rows, cols)
    return list(out) if isinstance(out, (list, tuple)) else [out]


def kernel(x, edge_index, Wf0, Wf1, Wf2, bf, Wb0, Wb1, Wb2, bb,
           Wl0, Wl1, Wl2, bl, g1, be1, g2, be2, g3, be3, p, Wg, bg):
    rows = edge_index[0]
    cols = edge_index[1]
    ones_c = jnp.ones((CHUNK, 128), jnp.float32)
    ones_n = jnp.ones((N, 128), jnp.float32)
    zeros_n = jnp.zeros((N, 128), jnp.float32)

    qdeg = _deg_kernel(ones_c, ones_n, zeros_n, cols)
    deg, t1 = _prep_call(qdeg[:N], qdeg[N:], x)

    [q1] = _run_prop(_prop1, [t1], zeros_n, rows, cols)
    (p1,), (t2,) = _scale_call([_halves(q1)], deg)
    [q2] = _run_prop(_prop1, [t2], zeros_n, rows, cols)
    m1, s1 = _mm_call(True, [x], [p1], [_halves(q2)], deg, Wf0, Wf1, Wf2, bf)
    t3 = _bn_call(m1, s1, deg, g1, be1)

    q3 = _run_prop(_prop3, t3, zeros_n, rows, cols)
    p3, t4 = _scale_call([_halves(q) for q in q3], deg)
    q4 = _run_prop(_prop3, t4, zeros_n, rows, cols)
    m2, s2 = _mm_call(False, t3, p3, [_halves(q) for q in q4], deg,
                      Wb0, Wb1, Wb2, bb)
    t5 = _bn_call(m2, s2, deg, g2, be2)

    q5 = _run_prop(_prop3, t5, zeros_n, rows, cols)
    p5, t6 = _scale_call([_halves(q) for q in q5], deg)
    q6 = _run_prop(_prop3, t6, zeros_n, rows, cols)
    m3, s3 = _mm_call(False, t5, p5, [_halves(q) for q in q6], deg,
                      Wl0, Wl1, Wl2, bl)

    logits, ypred = _final_call(m3, s3, g3, be3, p, Wg, bg)
    return (logits, ypred.reshape(-1))


# speedup vs baseline: 15.7039x; 1.0306x over previous
import functools

import jax
import jax.numpy as jnp
from jax import lax
from jax.experimental import pallas as pl
from jax.experimental.pallas import tpu as pltpu
from jax.experimental.pallas import tpu_sc as plsc

N = 10000
E = 320000
H = 128
EMB = 128
EPS = 1e-5
NS = 16
NC = 2
RPT = 624
TAIL0 = NS * RPT
TAIL = N - TAIL0
CHUNK = 80
GCH = 80
EPT = E // (NC * NS)
NCHF = EPT // GCH
BLK = 2000
NBLK = N // BLK

_mesh = plsc.VectorSubcoreMesh(core_axis_name="c", subcore_axis_name="s")


def _rows_copy(sid, src, dst, soff=0, doff=0):
    b = sid * RPT
    pltpu.sync_copy(src.at[pl.ds(soff + b, RPT)], dst.at[pl.ds(doff + b, RPT)])

    @pl.when(sid == 0)
    def _():
        pltpu.sync_copy(src.at[pl.ds(soff + TAIL0, TAIL)],
                        dst.at[pl.ds(doff + TAIL0, TAIL)])


def _make_prop(nparts):
    @functools.partial(
        pl.kernel,
        out_type=[jax.ShapeDtypeStruct((2 * N, 128), jnp.float32)
                  for _ in range(nparts)],
        mesh=_mesh,
        scratch_types=(
            [pltpu.VMEM((EPT,), jnp.int32)] +
            [pltpu.VMEM((GCH,), jnp.int32)] * 3 +
            [pltpu.VMEM((GCH, 128), jnp.float32)] * 3 +
            [pltpu.VMEM_SHARED((N, 128), jnp.float32)] +
            [pltpu.SemaphoreType.DMA] * 9
        ),
    )
    def prop(*refs):
        ts = refs[:nparts]
        zeros_hbm = refs[nparts]
        rows_hbm = refs[nparts + 1]
        cols_hbm = refs[nparts + 2]
        qs = refs[nparts + 3:nparts + 3 + nparts]
        sc = refs[nparts + 3 + nparts:]
        rows_all = sc[0]
        colbs = sc[1:4]
        gbs = sc[4:7]
        acc = sc[7]
        semgs = sc[8:11]
        semis = sc[11:14]
        semss = sc[14:17]
        cid = lax.axis_index("c")
        sid = lax.axis_index("s")
        base = cid * (E // NC) + sid * EPT
        banks = tuple(
            (colbs[b], gbs[b], semgs[b], semis[b], semss[b]) for b in range(3))

        pltpu.sync_copy(rows_hbm.at[pl.ds(base, EPT)], rows_all)

        for t_hbm, q_hbm in zip(ts, qs):
            @pl.when(cid == 0)
            def _():
                _rows_copy(sid, t_hbm, acc)

            @pl.when(cid == 1)
            def _():
                _rows_copy(sid, zeros_hbm, acc)

            plsc.subcore_barrier()

            def start(j, bank):
                colb, gb, semg, semi, sems = bank
                pltpu.async_copy(
                    cols_hbm.at[pl.ds(base + j * GCH, GCH)], colb, semi)
                pltpu.async_copy(
                    t_hbm.at[rows_all.at[pl.ds(j * GCH, GCH)]], gb, semg)

            def scat(j, bank):
                colb, gb, semg, semi, sems = bank
                pltpu.make_async_copy(
                    cols_hbm.at[pl.ds(base, GCH)], colb, semi).wait()
                pltpu.make_async_copy(
                    t_hbm.at[rows_all.at[pl.ds(0, GCH)]], gb, semg).wait()
                pltpu.async_copy(gb, acc.at[colb], sems, add=True)

            def waits(bank):
                colb, gb, semg, semi, sems = bank
                pltpu.make_async_copy(gb, acc.at[colb], sems).wait()

            start(0, banks[0])
            start(1, banks[1])
            scat(0, banks[0])
            start(2, banks[2])

            def body(jj, carry):
                k0 = 3 * jj + 1
                for o in range(3):
                    k = k0 + o
                    scat(k, banks[(1 + o) % 3])
                    waits(banks[o % 3])
                    start(k + 2, banks[o % 3])
                return carry

            lax.fori_loop(0, 40, body, 0)
            scat(121, banks[1])
            waits(banks[0])
            start(123, banks[0])
            scat(122, banks[2])
            waits(banks[1])
            start(124, banks[1])
            scat(123, banks[0])
            waits(banks[2])
            scat(124, banks[1])
            waits(banks[0])
            waits(banks[1])

            plsc.subcore_barrier()
            _rows_copy(sid, acc, q_hbm, doff=cid * N)

    return prop


_prop1 = _make_prop(1)
_prop3 = _make_prop(3)


@functools.partial(
    pl.kernel,
    out_type=jax.ShapeDtypeStruct((2 * N, 128), jnp.float32),
    mesh=_mesh,
    scratch_types=(
        [pltpu.VMEM((CHUNK, 128), jnp.float32)] +
        [pltpu.VMEM((CHUNK,), jnp.int32)] * 3 +
        [pltpu.VMEM_SHARED((N, 128), jnp.float32)] +
        [pltpu.SemaphoreType.DMA] * 6
    ),
)
def _deg_kernel(ones_c_hbm, ones_n_hbm, zeros_hbm, cols_hbm, q_hbm, *sc):
    obuf = sc[0]
    colbs = sc[1:4]
    acc = sc[4]
    semis = sc[5:8]
    semss = sc[8:11]
    cid = lax.axis_index("c")
    sid = lax.axis_index("s")
    base = cid * (E // NC) + sid * EPT
    banks = tuple((colbs[b], semis[b], semss[b]) for b in range(3))

    @pl.when(cid == 0)
    def _():
        _rows_copy(sid, ones_n_hbm, acc)

    @pl.when(cid == 1)
    def _():
        _rows_copy(sid, zeros_hbm, acc)

    pltpu.sync_copy(ones_c_hbm, obuf)
    plsc.subcore_barrier()

    def start(j, bank):
        colb, semi, sems = bank
        pltpu.async_copy(cols_hbm.at[pl.ds(base + j * CHUNK, CHUNK)],
                         colb, semi)

    def scat(j, bank):
        colb, semi, sems = bank
        pltpu.make_async_copy(
            cols_hbm.at[pl.ds(base, CHUNK)], colb, semi).wait()
        pltpu.async_copy(obuf, acc.at[colb], sems, add=True)

    def waits(bank):
        colb, semi, sems = bank
        pltpu.make_async_copy(obuf, acc.at[colb], sems).wait()

    start(0, banks[0])
    start(1, banks[1])
    scat(0, banks[0])
    start(2, banks[2])

    def body(jj, carry):
        k0 = 3 * jj + 1
        for o in range(3):
            k = k0 + o
            scat(k, banks[(1 + o) % 3])
            waits(banks[o % 3])
            start(k + 2, banks[o % 3])
        return carry

    lax.fori_loop(0, 40, body, 0)
    scat(121, banks[1])
    waits(banks[0])
    start(123, banks[0])
    scat(122, banks[2])
    waits(banks[1])
    start(124, banks[1])
    scat(123, banks[0])
    waits(banks[2])
    scat(124, banks[1])
    waits(banks[0])
    waits(banks[1])
    plsc.subcore_barrier()
    _rows_copy(sid, acc, q_hbm, doff=cid * N)


def _row_spec(cols):
    return pl.BlockSpec((BLK, cols), lambda i: (i, 0))


def _full_spec(r, c):
    return pl.BlockSpec((r, c), lambda i: (0, 0))


def _prep_body(qa_ref, qb_ref, x_ref, deg_ref, t1_ref):
    deg = qa_ref[...] + qb_ref[...]
    deg_ref[...] = deg[:, :16]
    t1_ref[...] = x_ref[...] * lax.rsqrt(deg[:, :1])


def _prep_call(qa, qb, x):
    return pl.pallas_call(
        _prep_body,
        grid=(NBLK,),
        in_specs=[_row_spec(128), _row_spec(128), _row_spec(128)],
        out_specs=[_row_spec(16), _row_spec(128)],
        out_shape=[jax.ShapeDtypeStruct((N, 16), jnp.float32),
                   jax.ShapeDtypeStruct((N, 128), jnp.float32)],
    )(qa, qb, x)


def _make_scale_body(nparts):
    def body(*refs):
        deg_ref = refs[2 * nparts]
        p_refs = refs[2 * nparts + 1:2 * nparts + 1 + nparts]
        t_refs = refs[2 * nparts + 1 + nparts:]
        r = 1.0 / deg_ref[:, :1]
        for k in range(nparts):
            psum = refs[2 * k][...] + refs[2 * k + 1][...]
            p_refs[k][...] = psum
            t_refs[k][...] = psum * r
    return body


def _scale_call(qparts, deg):
    nparts = len(qparts)
    flat = [a for pair in qparts for a in pair]
    shp = jax.ShapeDtypeStruct((N, 128), jnp.float32)
    out = pl.pallas_call(
        _make_scale_body(nparts),
        grid=(NBLK,),
        in_specs=[_row_spec(128)] * (2 * nparts) + [_row_spec(16)],
        out_specs=[_row_spec(128)] * (2 * nparts),
        out_shape=[shp] * (2 * nparts),
    )(*flat, deg)
    return out[:nparts], out[nparts:]


def _make_mm_body(first, nparts):
    def body(*refs):
        nh = 1 if first else nparts
        h0p = refs[:nh]
        p1p = refs[nh:nh + nparts]
        q2p = refs[nh + nparts:nh + nparts + 2 * nparts]
        deg_ref = refs[nh + 3 * nparts]
        w0_ref, w1_ref, w2_ref, b_ref = refs[nh + 3 * nparts + 1:
                                             nh + 3 * nparts + 5]
        m_ref, stats_ref = refs[nh + 3 * nparts + 5:]
        i = pl.program_id(0)
        deg = deg_ref[:, :1]
        dis = lax.rsqrt(deg)
        if first:
            h0 = h0p[0][...]
        else:
            h0 = jnp.concatenate([r[...] for r in h0p], axis=1) * jnp.sqrt(deg)
        p1 = jnp.concatenate([r[...] for r in p1p], axis=1) * dis
        p2 = jnp.concatenate(
            [q2p[2 * k][...] + q2p[2 * k + 1][...] for k in range(nparts)],
            axis=1) * dis
        dn = (((1,), (1,)), ((), ()))
        m0 = lax.dot_general(h0, w0_ref[...], dn,
                             preferred_element_type=jnp.float32)
        m1 = lax.dot_general(p1, w1_ref[...], dn,
                             preferred_element_type=jnp.float32)
        m2 = lax.dot_general(p2, w2_ref[...], dn,
                             preferred_element_type=jnp.float32)
        m = jnp.concatenate([m0, m1, m2], axis=1) + b_ref[...]
        m_ref[...] = m

        @pl.when(i == 0)
        def _():
            stats_ref[...] = jnp.zeros_like(stats_ref)

        stats_ref[0:1, :] += jnp.sum(m, axis=0, keepdims=True)
        stats_ref[1:2, :] += jnp.sum(m * m, axis=0, keepdims=True)

    return body


def _mm_call(first, h0parts, p1parts, q2parts, deg, w0, w1, w2, b):
    nparts = len(p1parts)
    din = w0.shape[1]
    q2flat = [a for pair in q2parts for a in pair]
    nin = len(h0parts) + nparts + 2 * nparts
    return pl.pallas_call(
        _make_mm_body(first, nparts),
        grid=(NBLK,),
        in_specs=[_row_spec(128)] * nin + [_row_spec(16)] +
                 [_full_spec(H, din)] * 3 + [_full_spec(1, 3 * H)],
        out_specs=[_row_spec(3 * H), _full_spec(8, 3 * H)],
        out_shape=[jax.ShapeDtypeStruct((N, 3 * H), jnp.float32),
                   jax.ShapeDtypeStruct((8, 3 * H), jnp.float32)],
    )(*h0parts, *p1parts, *q2flat, deg, w0, w1, w2, b.reshape(1, -1))


def _bn_body(m_ref, stats_ref, deg_ref, g_ref, be_ref, t0_ref, t1_ref, t2_ref):
    mu = stats_ref[0:1, :] * (1.0 / N)
    var = stats_ref[1:2, :] * (1.0 / N) - mu * mu
    y = (m_ref[...] - mu) * lax.rsqrt(var + EPS) * g_ref[...] + be_ref[...]
    y = jnp.where(y >= 0.0, y, 0.1 * y)
    t = y * lax.rsqrt(deg_ref[:, :1])
    t0_ref[...] = t[:, 0:128]
    t1_ref[...] = t[:, 128:256]
    t2_ref[...] = t[:, 256:384]


def _bn_call(m, stats, deg, g, be):
    shp = jax.ShapeDtypeStruct((N, 128), jnp.float32)
    return pl.pallas_call(
        _bn_body,
        grid=(NBLK,),
        in_specs=[_row_spec(3 * H), _full_spec(8, 3 * H), _row_spec(16),
                  _full_spec(1, 3 * H), _full_spec(1, 3 * H)],
        out_specs=[_row_spec(128)] * 3,
        out_shape=[shp] * 3,
    )(m, stats, deg, g.reshape(1, -1), be.reshape(1, -1))


def _safe_pow(x, p):
    safe = jnp.where(x > 0.0, x, 1.0)
    return jnp.where(x > 0.0, jnp.exp(p * jnp.log(safe)), 0.0)


def _final_body(m_ref, stats_ref, g_ref, be_ref, p_ref, wg_ref, bg_ref,
                logits_ref, ypred_ref, psum_ref):
    i = pl.program_id(0)
    mu = stats_ref[0:1, :] * (1.0 / N)
    var = stats_ref[1:2, :] * (1.0 / N) - mu * mu
    y = (m_ref[...] - mu) * lax.rsqrt(var + EPS) * g_ref[...] + be_ref[...]
    y = jnp.where(y >= 0.0, y, 0.1 * y)
    pp = p_ref[0, 0]
    s = _safe_pow(jnp.clip(y, 0.0, 100.0), pp)

    @pl.when(i == 0)
    def _():
        psum_ref[...] = jnp.zeros_like(psum_ref)

    psum_ref[0:1, :] += jnp.sum(s, axis=0, keepdims=True)

    @pl.when(i == NBLK - 1)
    def _():
        mean = psum_ref[0:1, :] * (1.0 / N)
        pooled = _safe_pow(jnp.clip(mean, 0.0, 100.0), 1.0 / pp)
        dn = (((1,), (1,)), ((), ()))
        logits = lax.dot_general(pooled, wg_ref[...], dn,
                                 preferred_element_type=jnp.float32)
        logits = logits + bg_ref[...]
        logits_ref[...] = logits
        iota = lax.broadcasted_iota(jnp.int32, (1, 10), 1)
        mx = jnp.max(logits, axis=1, keepdims=True)
        ypred_ref[...] = jnp.min(jnp.where(logits == mx, iota, 10),
                                 axis=1, keepdims=True)


def _final_call(m, stats, g, be, p, wg, bg):
    return pl.pallas_call(
        _final_body,
        grid=(NBLK,),
        in_specs=[_row_spec(3 * H), _full_spec(8, 3 * H),
                  _full_spec(1, 3 * H), _full_spec(1, 3 * H),
                  _full_spec(1, 1), _full_spec(10, 3 * EMB), _full_spec(1, 10)],
        out_specs=[_full_spec(1, 10), _full_spec(1, 1)],
        out_shape=[jax.ShapeDtypeStruct((1, 10), jnp.float32),
                   jax.ShapeDtypeStruct((1, 1), jnp.int32)],
        scratch_shapes=[pltpu.VMEM((8, 3 * H), jnp.float32)],
    )(m, stats, g.reshape(1, -1), be.reshape(1, -1), p.reshape(1, 1),
      wg, bg.reshape(1, -1))


def _halves(q):
    return (q[:N], q[N:])


def _run_prop(propfn, ts, zeros_n, rows, cols):
    out = propfn(*ts, zeros_n, rows, cols)
    return list(out) if isinstance(out, (list, tuple)) else [out]


def kernel(x, edge_index, Wf0, Wf1, Wf2, bf, Wb0, Wb1, Wb2, bb,
           Wl0, Wl1, Wl2, bl, g1, be1, g2, be2, g3, be3, p, Wg, bg):
    rows = edge_index[0]
    cols = edge_index[1]
    ones_c = jnp.ones((CHUNK, 128), jnp.float32)
    ones_n = jnp.ones((N, 128), jnp.float32)
    zeros_n = jnp.zeros((N, 128), jnp.float32)

    qdeg = _deg_kernel(ones_c, ones_n, zeros_n, cols)
    deg, t1 = _prep_call(qdeg[:N], qdeg[N:], x)

    [q1] = _run_prop(_prop1, [t1], zeros_n, rows, cols)
    (p1,), (t2,) = _scale_call([_halves(q1)], deg)
    [q2] = _run_prop(_prop1, [t2], zeros_n, rows, cols)
    m1, s1 = _mm_call(True, [x], [p1], [_halves(q2)], deg, Wf0, Wf1, Wf2, bf)
    t3 = _bn_call(m1, s1, deg, g1, be1)

    q3 = _run_prop(_prop3, t3, zeros_n, rows, cols)
    p3, t4 = _scale_call([_halves(q) for q in q3], deg)
    q4 = _run_prop(_prop3, t4, zeros_n, rows, cols)
    m2, s2 = _mm_call(False, t3, p3, [_halves(q) for q in q4], deg,
                      Wb0, Wb1, Wb2, bb)
    t5 = _bn_call(m2, s2, deg, g2, be2)

    q5 = _run_prop(_prop3, t5, zeros_n, rows, cols)
    p5, t6 = _scale_call([_halves(q) for q in q5], deg)
    q6 = _run_prop(_prop3, t6, zeros_n, rows, cols)
    m3, s3 = _mm_call(False, t5, p5, [_halves(q) for q in q6], deg,
                      Wl0, Wl1, Wl2, bl)

    logits, ypred = _final_call(m3, s3, g3, be3, p, Wg, bg)
    return (logits, ypred.reshape(-1))
